# baseline (speedup 1.0000x reference)
import jax
import jax.numpy as jnp
from jax import lax
from jax.experimental import pallas as pl
from jax.experimental.pallas import tpu as pltpu

N = 10000
E = 320000
HID = 128
G = 256
MAX_POS = 128
NUM_LB = 10
R = 3

BN = 2000
ECH = 512
NE = E // ECH


def _tc1_body(x_ref, se_ref, ce_ref, pe_ref, w_ref, rt_ref, b_ref, t_ref, or_ref):
    xb = x_ref[...]
    f32 = jnp.float32
    oh0 = (xb[:, 0:1] == lax.broadcasted_iota(jnp.int32, (1, 8), 1)).astype(f32)
    oh1 = (xb[:, 1:2] == lax.broadcasted_iota(jnp.int32, (1, 8), 1)).astype(f32)
    p2 = jnp.clip(xb[:, 2:3], 0, MAX_POS - 1)
    oh2 = (p2 == lax.broadcasted_iota(jnp.int32, (1, MAX_POS), 1)).astype(f32)
    h = jnp.concatenate([
        jnp.dot(oh0, se_ref[...], preferred_element_type=f32),
        jnp.dot(oh1, ce_ref[...], preferred_element_type=f32),
        jnp.dot(oh2, pe_ref[...], preferred_element_type=f32),
    ], axis=1)
    for r in range(R):
        t_ref[r] = jnp.dot(h, w_ref[r], preferred_element_type=f32)
    or_ref[...] = jnp.dot(h, rt_ref[...], preferred_element_type=f32) + b_ref[...]


def _tc1(x, se, ce, pe, W1, root1, b1):
    return pl.pallas_call(
        _tc1_body,
        grid=(N // BN,),
        in_specs=[
            pl.BlockSpec((BN, 3), lambda i: (i, 0)),
            pl.BlockSpec((8, 32), lambda i: (0, 0)),
            pl.BlockSpec((8, 32), lambda i: (0, 0)),
            pl.BlockSpec((MAX_POS, 32), lambda i: (0, 0)),
            pl.BlockSpec((R, 96, HID), lambda i: (0, 0, 0)),
            pl.BlockSpec((96, HID), lambda i: (0, 0)),
            pl.BlockSpec((1, HID), lambda i: (0, 0)),
        ],
        out_specs=[
            pl.BlockSpec((R, BN, HID), lambda i: (0, i, 0)),
            pl.BlockSpec((BN, HID), lambda i: (i, 0)),
        ],
        out_shape=[
            jax.ShapeDtypeStruct((R, N, HID), jnp.float32),
            jax.ShapeDtypeStruct((N, HID), jnp.float32),
        ],
    )(x, se, ce, pe, W1, root1, b1)


def _cnt_body(et_ref, dst_ref, cnt_ref):
    i = pl.program_id(0)

    @pl.when(i == 0)
    def _():
        cnt_ref[...] = jnp.zeros((R * N, 8), jnp.float32)

    one = jnp.ones((1, 8), jnp.float32)

    def body(k, c):
        kd = et_ref[k] * N + dst_ref[k]
        cnt_ref[pl.ds(kd, 1), :] = cnt_ref[pl.ds(kd, 1), :] + one
        return c
    lax.fori_loop(0, ECH, body, 0)


def _cnt(et, dst):
    return pl.pallas_call(
        _cnt_body,
        grid=(NE,),
        in_specs=[
            pl.BlockSpec((ECH,), lambda i: (i,), memory_space=pltpu.SMEM),
            pl.BlockSpec((ECH,), lambda i: (i,), memory_space=pltpu.SMEM),
        ],
        out_specs=pl.BlockSpec((R * N, 8), lambda i: (0, 0)),
        out_shape=jax.ShapeDtypeStruct((R * N, 8), jnp.float32),
    )(et, dst)


def _msg_body(et_ref, src_ref, dst_ref, t_ref, acc_ref):
    i = pl.program_id(0)

    @pl.when(i == 0)
    def _():
        acc_ref[...] = jnp.zeros((R * N, HID), jnp.float32)

    def body(k, c):
        r = et_ref[k]
        ks = r * N + src_ref[k]
        kd = r * N + dst_ref[k]
        acc_ref[pl.ds(kd, 1), :] = (acc_ref[pl.ds(kd, 1), :]
                                    + t_ref[pl.ds(ks, 1), :])
        return c
    lax.fori_loop(0, ECH, body, 0)


def _msg(et, src, dst, T):
    return pl.pallas_call(
        _msg_body,
        grid=(NE,),
        in_specs=[
            pl.BlockSpec((ECH,), lambda i: (i,), memory_space=pltpu.SMEM),
            pl.BlockSpec((ECH,), lambda i: (i,), memory_space=pltpu.SMEM),
            pl.BlockSpec((ECH,), lambda i: (i,), memory_space=pltpu.SMEM),
            pl.BlockSpec((R * N, HID), lambda i: (0, 0)),
        ],
        out_specs=pl.BlockSpec((R * N, HID), lambda i: (0, 0)),
        out_shape=jax.ShapeDtypeStruct((R * N, HID), jnp.float32),
    )(et, src, dst, T)


def _tc2_body(or_ref, a_ref, c_ref, w_ref, rt_ref, b_ref, t_ref, or2_ref):
    f32 = jnp.float32
    h = or_ref[...]
    for r in range(R):
        inv = 1.0 / jnp.maximum(c_ref[r, :, 0:1], 1.0)
        h = h + a_ref[r] * inv
    h = jnp.maximum(h, 0.0)
    for r in range(R):
        t_ref[r] = jnp.dot(h, w_ref[r], preferred_element_type=f32)
    or2_ref[...] = jnp.dot(h, rt_ref[...], preferred_element_type=f32) + b_ref[...]


def _tc2(or1, A, C, W2, root2, b2):
    return pl.pallas_call(
        _tc2_body,
        grid=(N // BN,),
        in_specs=[
            pl.BlockSpec((BN, HID), lambda i: (i, 0)),
            pl.BlockSpec((R, BN, HID), lambda i: (0, i, 0)),
            pl.BlockSpec((R, BN, 8), lambda i: (0, i, 0)),
            pl.BlockSpec((R, HID, HID), lambda i: (0, 0, 0)),
            pl.BlockSpec((HID, HID), lambda i: (0, 0)),
            pl.BlockSpec((1, HID), lambda i: (0, 0)),
        ],
        out_specs=[
            pl.BlockSpec((R, BN, HID), lambda i: (0, i, 0)),
            pl.BlockSpec((BN, HID), lambda i: (i, 0)),
        ],
        out_shape=[
            jax.ShapeDtypeStruct((R, N, HID), jnp.float32),
            jax.ShapeDtypeStruct((N, HID), jnp.float32),
        ],
    )(or1, A, C, W2, root2, b2)


def _tc3_body(or_ref, a_ref, c_ref, batch_ref, cw_ref, cb_ref, out_ref,
              hs_acc, cnt_acc):
    i = pl.program_id(0)
    f32 = jnp.float32

    @pl.when(i == 0)
    def _():
        hs_acc[...] = jnp.zeros((G, HID), f32)
        cnt_acc[...] = jnp.zeros((G, HID), f32)

    h = or_ref[...]
    for r in range(R):
        inv = 1.0 / jnp.maximum(c_ref[r, :, 0:1], 1.0)
        h = h + a_ref[r] * inv
    h = jnp.maximum(h, 0.0)
    oh = (batch_ref[...] == lax.broadcasted_iota(jnp.int32, (1, G), 1)).astype(f32)
    dn = (((0,), (0,)), ((), ()))
    hs_acc[...] += lax.dot_general(oh, h, dn, preferred_element_type=f32)
    cnt_acc[...] += lax.dot_general(oh, jnp.ones((BN, HID), f32), dn,
                                    preferred_element_type=f32)

    @pl.when(i == (N // BN) - 1)
    def _():
        hg = hs_acc[...] / jnp.maximum(cnt_acc[...], 1.0)
        out_ref[...] = jnp.dot(hg, cw_ref[...], preferred_element_type=f32) + cb_ref[...]


def _tc3(or2, A, C, batch2d, cls_W, cls_b):
    return pl.pallas_call(
        _tc3_body,
        grid=(N // BN,),
        in_specs=[
            pl.BlockSpec((BN, HID), lambda i: (i, 0)),
            pl.BlockSpec((R, BN, HID), lambda i: (0, i, 0)),
            pl.BlockSpec((R, BN, 8), lambda i: (0, i, 0)),
            pl.BlockSpec((BN, 1), lambda i: (i, 0)),
            pl.BlockSpec((HID, NUM_LB), lambda i: (0, 0)),
            pl.BlockSpec((1, NUM_LB), lambda i: (0, 0)),
        ],
        out_specs=pl.BlockSpec((G, NUM_LB), lambda i: (0, 0)),
        out_shape=jax.ShapeDtypeStruct((G, NUM_LB), jnp.float32),
        scratch_shapes=[
            pltpu.VMEM((G, HID), jnp.float32),
            pltpu.VMEM((G, HID), jnp.float32),
        ],
    )(or2, A, C, batch2d, cls_W, cls_b)


def kernel(x, edge_index, edge_type, batch, shape_emb, color_emb, pos_emb,
           W1, root1, b1, W2, root2, b2, cls_W, cls_b):
    src = edge_index[0]
    dst = edge_index[1]
    et = edge_type

    T1, or1 = _tc1(x, shape_emb, color_emb, pos_emb, W1, root1,
                   b1.reshape(1, HID))
    cnt = _cnt(et, dst).reshape(R, N, 8)
    A1 = _msg(et, src, dst, T1.reshape(R * N, HID)).reshape(R, N, HID)
    T2, or2 = _tc2(or1, A1, cnt, W2, root2, b2.reshape(1, HID))
    A2 = _msg(et, src, dst, T2.reshape(R * N, HID)).reshape(R, N, HID)
    return _tc3(or2, A2, cnt, batch.reshape(N, 1), cls_W,
                cls_b.reshape(1, NUM_LB))


# speedup vs baseline: 2.0835x; 2.0835x over previous
import jax
import jax.numpy as jnp
from jax import lax
from jax.experimental import pallas as pl
from jax.experimental.pallas import tpu as pltpu

N = 10000
E = 320000
HID = 128
G = 256
MAX_POS = 128
NUM_LB = 10
R = 3

BN = 2000
NC = 2
ECH = 256
NEH = E // (NC * ECH)

_par = pltpu.CompilerParams(dimension_semantics=("parallel", "arbitrary"))


def _tc1_body(x_ref, se_ref, ce_ref, pe_ref, w_ref, rt_ref, b_ref, t_ref, or_ref):
    xb = x_ref[...]
    f32 = jnp.float32
    oh0 = (xb[:, 0:1] == lax.broadcasted_iota(jnp.int32, (1, 8), 1)).astype(f32)
    oh1 = (xb[:, 1:2] == lax.broadcasted_iota(jnp.int32, (1, 8), 1)).astype(f32)
    p2 = jnp.clip(xb[:, 2:3], 0, MAX_POS - 1)
    oh2 = (p2 == lax.broadcasted_iota(jnp.int32, (1, MAX_POS), 1)).astype(f32)
    h = jnp.concatenate([
        jnp.dot(oh0, se_ref[...], preferred_element_type=f32),
        jnp.dot(oh1, ce_ref[...], preferred_element_type=f32),
        jnp.dot(oh2, pe_ref[...], preferred_element_type=f32),
    ], axis=1)
    for r in range(R):
        t_ref[r] = jnp.dot(h, w_ref[r], preferred_element_type=f32)
    or_ref[...] = jnp.dot(h, rt_ref[...], preferred_element_type=f32) + b_ref[...]


def _tc1(x, se, ce, pe, W1, root1, b1):
    return pl.pallas_call(
        _tc1_body,
        grid=(N // BN,),
        in_specs=[
            pl.BlockSpec((BN, 3), lambda i: (i, 0)),
            pl.BlockSpec((8, 32), lambda i: (0, 0)),
            pl.BlockSpec((8, 32), lambda i: (0, 0)),
            pl.BlockSpec((MAX_POS, 32), lambda i: (0, 0)),
            pl.BlockSpec((R, 96, HID), lambda i: (0, 0, 0)),
            pl.BlockSpec((96, HID), lambda i: (0, 0)),
            pl.BlockSpec((1, HID), lambda i: (0, 0)),
        ],
        out_specs=[
            pl.BlockSpec((R, BN, HID), lambda i: (0, i, 0)),
            pl.BlockSpec((BN, HID), lambda i: (i, 0)),
        ],
        out_shape=[
            jax.ShapeDtypeStruct((R, N, HID), jnp.float32),
            jax.ShapeDtypeStruct((N, HID), jnp.float32),
        ],
    )(x, se, ce, pe, W1, root1, b1)


def _msgc_body(et_ref, src_ref, dst_ref, t_ref, accO, cntO,
               acc, cnt, sem1, sem2):
    c = pl.program_id(0)
    j = pl.program_id(1)

    @pl.when(j == 0)
    def _():
        acc[...] = jnp.zeros((R * N, HID), jnp.float32)
        cnt[...] = jnp.zeros((R * N, 8), jnp.float32)

    one = jnp.ones((1, 8), jnp.float32)

    def body(k, cc):
        r = et_ref[k]
        ks = r * N + src_ref[k]
        kd = r * N + dst_ref[k]
        acc[pl.ds(kd, 1), :] = acc[pl.ds(kd, 1), :] + t_ref[pl.ds(ks, 1), :]
        cnt[pl.ds(kd, 1), :] = cnt[pl.ds(kd, 1), :] + one
        return cc
    lax.fori_loop(0, ECH, body, 0, unroll=2)

    @pl.when(j == NEH - 1)
    def _():
        cp1 = pltpu.make_async_copy(acc, accO.at[c], sem1)
        cp2 = pltpu.make_async_copy(cnt, cntO.at[c], sem2)
        cp1.start()
        cp2.start()
        cp1.wait()
        cp2.wait()


def _msgc(et, src, dst, T):
    return pl.pallas_call(
        _msgc_body,
        grid=(NC, NEH),
        in_specs=[
            pl.BlockSpec((ECH,), lambda c, j: (c * NEH + j,), memory_space=pltpu.SMEM),
            pl.BlockSpec((ECH,), lambda c, j: (c * NEH + j,), memory_space=pltpu.SMEM),
            pl.BlockSpec((ECH,), lambda c, j: (c * NEH + j,), memory_space=pltpu.SMEM),
            pl.BlockSpec((R * N, HID), lambda c, j: (0, 0)),
        ],
        out_specs=[
            pl.BlockSpec(memory_space=pl.ANY),
            pl.BlockSpec(memory_space=pl.ANY),
        ],
        out_shape=[
            jax.ShapeDtypeStruct((NC, R * N, HID), jnp.float32),
            jax.ShapeDtypeStruct((NC, R * N, 8), jnp.float32),
        ],
        scratch_shapes=[
            pltpu.VMEM((R * N, HID), jnp.float32),
            pltpu.VMEM((R * N, 8), jnp.float32),
            pltpu.SemaphoreType.DMA,
            pltpu.SemaphoreType.DMA,
        ],
        compiler_params=_par,
    )(et, src, dst, T)


def _msg_body(et_ref, src_ref, dst_ref, t_ref, accO, acc, sem1):
    c = pl.program_id(0)
    j = pl.program_id(1)

    @pl.when(j == 0)
    def _():
        acc[...] = jnp.zeros((R * N, HID), jnp.float32)

    def body(k, cc):
        r = et_ref[k]
        ks = r * N + src_ref[k]
        kd = r * N + dst_ref[k]
        acc[pl.ds(kd, 1), :] = acc[pl.ds(kd, 1), :] + t_ref[pl.ds(ks, 1), :]
        return cc
    lax.fori_loop(0, ECH, body, 0, unroll=2)

    @pl.when(j == NEH - 1)
    def _():
        cp1 = pltpu.make_async_copy(acc, accO.at[c], sem1)
        cp1.start()
        cp1.wait()


def _msg(et, src, dst, T):
    return pl.pallas_call(
        _msg_body,
        grid=(NC, NEH),
        in_specs=[
            pl.BlockSpec((ECH,), lambda c, j: (c * NEH + j,), memory_space=pltpu.SMEM),
            pl.BlockSpec((ECH,), lambda c, j: (c * NEH + j,), memory_space=pltpu.SMEM),
            pl.BlockSpec((ECH,), lambda c, j: (c * NEH + j,), memory_space=pltpu.SMEM),
            pl.BlockSpec((R * N, HID), lambda c, j: (0, 0)),
        ],
        out_specs=pl.BlockSpec(memory_space=pl.ANY),
        out_shape=jax.ShapeDtypeStruct((NC, R * N, HID), jnp.float32),
        scratch_shapes=[
            pltpu.VMEM((R * N, HID), jnp.float32),
            pltpu.SemaphoreType.DMA,
        ],
        compiler_params=_par,
    )(et, src, dst, T)


def _tc2_body(or_ref, a_ref, c_ref, w_ref, rt_ref, b_ref, t_ref, or2_ref):
    f32 = jnp.float32
    h = or_ref[...]
    for r in range(R):
        cr = c_ref[0, r, :, 0:1] + c_ref[1, r, :, 0:1]
        inv = 1.0 / jnp.maximum(cr, 1.0)
        h = h + (a_ref[0, r] + a_ref[1, r]) * inv
    h = jnp.maximum(h, 0.0)
    for r in range(R):
        t_ref[r] = jnp.dot(h, w_ref[r], preferred_element_type=f32)
    or2_ref[...] = jnp.dot(h, rt_ref[...], preferred_element_type=f32) + b_ref[...]


def _tc2(or1, A, C, W2, root2, b2):
    return pl.pallas_call(
        _tc2_body,
        grid=(N // BN,),
        in_specs=[
            pl.BlockSpec((BN, HID), lambda i: (i, 0)),
            pl.BlockSpec((NC, R, BN, HID), lambda i: (0, 0, i, 0)),
            pl.BlockSpec((NC, R, BN, 8), lambda i: (0, 0, i, 0)),
            pl.BlockSpec((R, HID, HID), lambda i: (0, 0, 0)),
            pl.BlockSpec((HID, HID), lambda i: (0, 0)),
            pl.BlockSpec((1, HID), lambda i: (0, 0)),
        ],
        out_specs=[
            pl.BlockSpec((R, BN, HID), lambda i: (0, i, 0)),
            pl.BlockSpec((BN, HID), lambda i: (i, 0)),
        ],
        out_shape=[
            jax.ShapeDtypeStruct((R, N, HID), jnp.float32),
            jax.ShapeDtypeStruct((N, HID), jnp.float32),
        ],
    )(or1, A, C, W2, root2, b2)


def _tc3_body(or_ref, a_ref, c_ref, batch_ref, cw_ref, cb_ref, out_ref,
              hs_acc, cnt_acc):
    i = pl.program_id(0)
    f32 = jnp.float32

    @pl.when(i == 0)
    def _():
        hs_acc[...] = jnp.zeros((G, HID), f32)
        cnt_acc[...] = jnp.zeros((G, HID), f32)

    h = or_ref[...]
    for r in range(R):
        cr = c_ref[0, r, :, 0:1] + c_ref[1, r, :, 0:1]
        inv = 1.0 / jnp.maximum(cr, 1.0)
        h = h + (a_ref[0, r] + a_ref[1, r]) * inv
    h = jnp.maximum(h, 0.0)
    oh = (batch_ref[...] == lax.broadcasted_iota(jnp.int32, (1, G), 1)).astype(f32)
    dn = (((0,), (0,)), ((), ()))
    hs_acc[...] += lax.dot_general(oh, h, dn, preferred_element_type=f32)
    cnt_acc[...] += lax.dot_general(oh, jnp.ones((BN, HID), f32), dn,
                                    preferred_element_type=f32)

    @pl.when(i == (N // BN) - 1)
    def _():
        hg = hs_acc[...] / jnp.maximum(cnt_acc[...], 1.0)
        out_ref[...] = jnp.dot(hg, cw_ref[...], preferred_element_type=f32) + cb_ref[...]


def _tc3(or2, A, C, batch2d, cls_W, cls_b):
    return pl.pallas_call(
        _tc3_body,
        grid=(N // BN,),
        in_specs=[
            pl.BlockSpec((BN, HID), lambda i: (i, 0)),
            pl.BlockSpec((NC, R, BN, HID), lambda i: (0, 0, i, 0)),
            pl.BlockSpec((NC, R, BN, 8), lambda i: (0, 0, i, 0)),
            pl.BlockSpec((BN, 1), lambda i: (i, 0)),
            pl.BlockSpec((HID, NUM_LB), lambda i: (0, 0)),
            pl.BlockSpec((1, NUM_LB), lambda i: (0, 0)),
        ],
        out_specs=pl.BlockSpec((G, NUM_LB), lambda i: (0, 0)),
        out_shape=jax.ShapeDtypeStruct((G, NUM_LB), jnp.float32),
        scratch_shapes=[
            pltpu.VMEM((G, HID), jnp.float32),
            pltpu.VMEM((G, HID), jnp.float32),
        ],
    )(or2, A, C, batch2d, cls_W, cls_b)


def kernel(x, edge_index, edge_type, batch, shape_emb, color_emb, pos_emb,
           W1, root1, b1, W2, root2, b2, cls_W, cls_b):
    src = edge_index[0]
    dst = edge_index[1]
    et = edge_type

    T1, or1 = _tc1(x, shape_emb, color_emb, pos_emb, W1, root1,
                   b1.reshape(1, HID))
    A1, cnt = _msgc(et, src, dst, T1.reshape(R * N, HID))
    A1 = A1.reshape(NC, R, N, HID)
    cnt = cnt.reshape(NC, R, N, 8)
    T2, or2 = _tc2(or1, A1, cnt, W2, root2, b2.reshape(1, HID))
    A2 = _msg(et, src, dst, T2.reshape(R * N, HID)).reshape(NC, R, N, HID)
    return _tc3(or2, A2, cnt, batch.reshape(N, 1), cls_W,
                cls_b.reshape(1, NUM_LB))


# speedup vs baseline: 2.5332x; 1.2158x over previous
import jax
import jax.numpy as jnp
from jax import lax
from jax.experimental import pallas as pl
from jax.experimental.pallas import tpu as pltpu

N = 10000
E = 320000
HID = 128
G = 256
MAX_POS = 128
NUM_LB = 10
R = 3

BN = 2000
NC = 2
ECH = 256
NEH = E // (NC * ECH)

_par = pltpu.CompilerParams(dimension_semantics=("parallel", "arbitrary"))


def _tc1_body(x_ref, se_ref, ce_ref, pe_ref, w_ref, rt_ref, b_ref, t_ref, or_ref):
    xb = x_ref[...]
    f32 = jnp.float32
    oh0 = (xb[:, 0:1] == lax.broadcasted_iota(jnp.int32, (1, 8), 1)).astype(f32)
    oh1 = (xb[:, 1:2] == lax.broadcasted_iota(jnp.int32, (1, 8), 1)).astype(f32)
    p2 = jnp.clip(xb[:, 2:3], 0, MAX_POS - 1)
    oh2 = (p2 == lax.broadcasted_iota(jnp.int32, (1, MAX_POS), 1)).astype(f32)
    h = jnp.concatenate([
        jnp.dot(oh0, se_ref[...], preferred_element_type=f32),
        jnp.dot(oh1, ce_ref[...], preferred_element_type=f32),
        jnp.dot(oh2, pe_ref[...], preferred_element_type=f32),
    ], axis=1)
    for r in range(R):
        t_ref[r] = jnp.dot(h, w_ref[r], preferred_element_type=f32)
    or_ref[...] = jnp.dot(h, rt_ref[...], preferred_element_type=f32) + b_ref[...]


def _tc1(x, se, ce, pe, W1, root1, b1):
    return pl.pallas_call(
        _tc1_body,
        grid=(N // BN,),
        in_specs=[
            pl.BlockSpec((BN, 3), lambda i: (i, 0)),
            pl.BlockSpec((8, 32), lambda i: (0, 0)),
            pl.BlockSpec((8, 32), lambda i: (0, 0)),
            pl.BlockSpec((MAX_POS, 32), lambda i: (0, 0)),
            pl.BlockSpec((R, 96, HID), lambda i: (0, 0, 0)),
            pl.BlockSpec((96, HID), lambda i: (0, 0)),
            pl.BlockSpec((1, HID), lambda i: (0, 0)),
        ],
        out_specs=[
            pl.BlockSpec((R, BN, HID), lambda i: (0, i, 0)),
            pl.BlockSpec((BN, HID), lambda i: (i, 0)),
        ],
        out_shape=[
            jax.ShapeDtypeStruct((R, N, HID), jnp.float32),
            jax.ShapeDtypeStruct((N, HID), jnp.float32),
        ],
    )(x, se, ce, pe, W1, root1, b1)


def _msgc_body(et_ref, src_ref, dst_ref, t_ref, accO, cntO,
               acc_a, acc_b, cnt, sem1, sem2, sem3):
    c = pl.program_id(0)
    j = pl.program_id(1)

    @pl.when(j == 0)
    def _():
        acc_a[...] = jnp.zeros((R * N, HID), jnp.float32)
        acc_b[...] = jnp.zeros((R * N, HID), jnp.float32)
        cnt[...] = jnp.zeros((R * N, 8), jnp.float32)

    one = jnp.ones((1, 8), jnp.float32)

    def body(k, cc):
        ka = 2 * k
        kb = 2 * k + 1
        ra = et_ref[ka]
        rb = et_ref[kb]
        ksa = ra * N + src_ref[ka]
        ksb = rb * N + src_ref[kb]
        kda = ra * N + dst_ref[ka]
        kdb = rb * N + dst_ref[kb]
        acc_a[pl.ds(kda, 1), :] = (acc_a[pl.ds(kda, 1), :]
                                   + t_ref[pl.ds(ksa, 1), :])
        acc_b[pl.ds(kdb, 1), :] = (acc_b[pl.ds(kdb, 1), :]
                                   + t_ref[pl.ds(ksb, 1), :])
        cnt[pl.ds(kda, 1), :] = cnt[pl.ds(kda, 1), :] + one
        cnt[pl.ds(kdb, 1), :] = cnt[pl.ds(kdb, 1), :] + one
        return cc
    lax.fori_loop(0, ECH // 2, body, 0, unroll=2)

    @pl.when(j == NEH - 1)
    def _():
        cp1 = pltpu.make_async_copy(acc_a, accO.at[c, 0], sem1)
        cp2 = pltpu.make_async_copy(acc_b, accO.at[c, 1], sem2)
        cp3 = pltpu.make_async_copy(cnt, cntO.at[c], sem3)
        cp1.start()
        cp2.start()
        cp3.start()
        cp1.wait()
        cp2.wait()
        cp3.wait()


def _msgc(et, src, dst, T):
    return pl.pallas_call(
        _msgc_body,
        grid=(NC, NEH),
        in_specs=[
            pl.BlockSpec((ECH,), lambda c, j: (c * NEH + j,), memory_space=pltpu.SMEM),
            pl.BlockSpec((ECH,), lambda c, j: (c * NEH + j,), memory_space=pltpu.SMEM),
            pl.BlockSpec((ECH,), lambda c, j: (c * NEH + j,), memory_space=pltpu.SMEM),
            pl.BlockSpec((R * N, HID), lambda c, j: (0, 0)),
        ],
        out_specs=[
            pl.BlockSpec(memory_space=pl.ANY),
            pl.BlockSpec(memory_space=pl.ANY),
        ],
        out_shape=[
            jax.ShapeDtypeStruct((NC, 2, R * N, HID), jnp.float32),
            jax.ShapeDtypeStruct((NC, R * N, 8), jnp.float32),
        ],
        scratch_shapes=[
            pltpu.VMEM((R * N, HID), jnp.float32),
            pltpu.VMEM((R * N, HID), jnp.float32),
            pltpu.VMEM((R * N, 8), jnp.float32),
            pltpu.SemaphoreType.DMA,
            pltpu.SemaphoreType.DMA,
            pltpu.SemaphoreType.DMA,
        ],
        compiler_params=_par,
    )(et, src, dst, T)


def _msg_body(et_ref, src_ref, dst_ref, t_ref, accO, acc_a, acc_b, sem1, sem2):
    c = pl.program_id(0)
    j = pl.program_id(1)

    @pl.when(j == 0)
    def _():
        acc_a[...] = jnp.zeros((R * N, HID), jnp.float32)
        acc_b[...] = jnp.zeros((R * N, HID), jnp.float32)

    def body(k, cc):
        ka = 2 * k
        kb = 2 * k + 1
        ra = et_ref[ka]
        rb = et_ref[kb]
        ksa = ra * N + src_ref[ka]
        ksb = rb * N + src_ref[kb]
        kda = ra * N + dst_ref[ka]
        kdb = rb * N + dst_ref[kb]
        acc_a[pl.ds(kda, 1), :] = (acc_a[pl.ds(kda, 1), :]
                                   + t_ref[pl.ds(ksa, 1), :])
        acc_b[pl.ds(kdb, 1), :] = (acc_b[pl.ds(kdb, 1), :]
                                   + t_ref[pl.ds(ksb, 1), :])
        return cc
    lax.fori_loop(0, ECH // 2, body, 0, unroll=2)

    @pl.when(j == NEH - 1)
    def _():
        cp1 = pltpu.make_async_copy(acc_a, accO.at[c, 0], sem1)
        cp2 = pltpu.make_async_copy(acc_b, accO.at[c, 1], sem2)
        cp1.start()
        cp2.start()
        cp1.wait()
        cp2.wait()


def _msg(et, src, dst, T):
    return pl.pallas_call(
        _msg_body,
        grid=(NC, NEH),
        in_specs=[
            pl.BlockSpec((ECH,), lambda c, j: (c * NEH + j,), memory_space=pltpu.SMEM),
            pl.BlockSpec((ECH,), lambda c, j: (c * NEH + j,), memory_space=pltpu.SMEM),
            pl.BlockSpec((ECH,), lambda c, j: (c * NEH + j,), memory_space=pltpu.SMEM),
            pl.BlockSpec((R * N, HID), lambda c, j: (0, 0)),
        ],
        out_specs=pl.BlockSpec(memory_space=pl.ANY),
        out_shape=jax.ShapeDtypeStruct((NC, 2, R * N, HID), jnp.float32),
        scratch_shapes=[
            pltpu.VMEM((R * N, HID), jnp.float32),
            pltpu.VMEM((R * N, HID), jnp.float32),
            pltpu.SemaphoreType.DMA,
            pltpu.SemaphoreType.DMA,
        ],
        compiler_params=_par,
    )(et, src, dst, T)


def _tc2_body(or_ref, a_ref, c_ref, w_ref, rt_ref, b_ref, t_ref, or2_ref):
    f32 = jnp.float32
    h = or_ref[...]
    for r in range(R):
        cr = c_ref[0, r, :, 0:1] + c_ref[1, r, :, 0:1]
        inv = 1.0 / jnp.maximum(cr, 1.0)
        h = h + (a_ref[0, r] + a_ref[1, r]
                 + a_ref[2, r] + a_ref[3, r]) * inv
    h = jnp.maximum(h, 0.0)
    for r in range(R):
        t_ref[r] = jnp.dot(h, w_ref[r], preferred_element_type=f32)
    or2_ref[...] = jnp.dot(h, rt_ref[...], preferred_element_type=f32) + b_ref[...]


def _tc2(or1, A, C, W2, root2, b2):
    return pl.pallas_call(
        _tc2_body,
        grid=(N // BN,),
        in_specs=[
            pl.BlockSpec((BN, HID), lambda i: (i, 0)),
            pl.BlockSpec((2 * NC, R, BN, HID), lambda i: (0, 0, i, 0)),
            pl.BlockSpec((NC, R, BN, 8), lambda i: (0, 0, i, 0)),
            pl.BlockSpec((R, HID, HID), lambda i: (0, 0, 0)),
            pl.BlockSpec((HID, HID), lambda i: (0, 0)),
            pl.BlockSpec((1, HID), lambda i: (0, 0)),
        ],
        out_specs=[
            pl.BlockSpec((R, BN, HID), lambda i: (0, i, 0)),
            pl.BlockSpec((BN, HID), lambda i: (i, 0)),
        ],
        out_shape=[
            jax.ShapeDtypeStruct((R, N, HID), jnp.float32),
            jax.ShapeDtypeStruct((N, HID), jnp.float32),
        ],
    )(or1, A, C, W2, root2, b2)


def _tc3_body(or_ref, a_ref, c_ref, batch_ref, cw_ref, cb_ref, out_ref,
              hs_acc, cnt_acc):
    i = pl.program_id(0)
    f32 = jnp.float32

    @pl.when(i == 0)
    def _():
        hs_acc[...] = jnp.zeros((G, HID), f32)
        cnt_acc[...] = jnp.zeros((G, HID), f32)

    h = or_ref[...]
    for r in range(R):
        cr = c_ref[0, r, :, 0:1] + c_ref[1, r, :, 0:1]
        inv = 1.0 / jnp.maximum(cr, 1.0)
        h = h + (a_ref[0, r] + a_ref[1, r]
                 + a_ref[2, r] + a_ref[3, r]) * inv
    h = jnp.maximum(h, 0.0)
    oh = (batch_ref[...] == lax.broadcasted_iota(jnp.int32, (1, G), 1)).astype(f32)
    dn = (((0,), (0,)), ((), ()))
    hs_acc[...] += lax.dot_general(oh, h, dn, preferred_element_type=f32)
    cnt_acc[...] += lax.dot_general(oh, jnp.ones((BN, HID), f32), dn,
                                    preferred_element_type=f32)

    @pl.when(i == (N // BN) - 1)
    def _():
        hg = hs_acc[...] / jnp.maximum(cnt_acc[...], 1.0)
        out_ref[...] = jnp.dot(hg, cw_ref[...], preferred_element_type=f32) + cb_ref[...]


def _tc3(or2, A, C, batch2d, cls_W, cls_b):
    return pl.pallas_call(
        _tc3_body,
        grid=(N // BN,),
        in_specs=[
            pl.BlockSpec((BN, HID), lambda i: (i, 0)),
            pl.BlockSpec((2 * NC, R, BN, HID), lambda i: (0, 0, i, 0)),
            pl.BlockSpec((NC, R, BN, 8), lambda i: (0, 0, i, 0)),
            pl.BlockSpec((BN, 1), lambda i: (i, 0)),
            pl.BlockSpec((HID, NUM_LB), lambda i: (0, 0)),
            pl.BlockSpec((1, NUM_LB), lambda i: (0, 0)),
        ],
        out_specs=pl.BlockSpec((G, NUM_LB), lambda i: (0, 0)),
        out_shape=jax.ShapeDtypeStruct((G, NUM_LB), jnp.float32),
        scratch_shapes=[
            pltpu.VMEM((G, HID), jnp.float32),
            pltpu.VMEM((G, HID), jnp.float32),
        ],
    )(or2, A, C, batch2d, cls_W, cls_b)


def kernel(x, edge_index, edge_type, batch, shape_emb, color_emb, pos_emb,
           W1, root1, b1, W2, root2, b2, cls_W, cls_b):
    src = edge_index[0]
    dst = edge_index[1]
    et = edge_type

    T1, or1 = _tc1(x, shape_emb, color_emb, pos_emb, W1, root1,
                   b1.reshape(1, HID))
    A1, cnt = _msgc(et, src, dst, T1.reshape(R * N, HID))
    A1 = A1.reshape(2 * NC, R, N, HID)
    cnt = cnt.reshape(NC, R, N, 8)
    T2, or2 = _tc2(or1, A1, cnt, W2, root2, b2.reshape(1, HID))
    A2 = _msg(et, src, dst, T2.reshape(R * N, HID)).reshape(2 * NC, R, N, HID)
    return _tc3(or2, A2, cnt, batch.reshape(N, 1), cls_W,
                cls_b.reshape(1, NUM_LB))
